# Initial kernel scaffold; baseline (speedup 1.0000x reference)
import jax
import jax.numpy as jnp
from jax.experimental import pallas as pl


def kernel(X, missing_pattern, view_W, view_b, W1, att_src1, att_dst1, b1, W2, att_src2, att_dst2, b2, fc_W, fc_b):
    raise NotImplementedError("write your pallas kernel here")


# speedup vs baseline: 32.5666x; 32.5666x over previous
import functools

import jax
import jax.numpy as jnp
from jax.experimental import pallas as pl

_NEG = -1e30


def _shift(a, d):
    if d == 0:
        return a
    return jnp.concatenate([a[d:], a[:d]], axis=0)


def _head_sums(prod, heads, ch):
    parts = [jnp.sum(prod[:, i * ch:(i + 1) * ch], axis=1, keepdims=True)
             for i in range(heads)]
    return parts[0] if heads == 1 else jnp.concatenate(parts, axis=1)


def _gat_layer(h, a_src_vec, a_dst_vec, kio, mp, own_bit, heads, ch):
    asrc = _head_sums(h * a_src_vec, heads, ch)
    adst = _head_sums(h * a_dst_vec, heads, ch)

    sh_as = {d: _shift(asrc, d) for d in range(-3, 4)}

    ems = []
    for j in range(4):
        ab = None
        for k in range(4):
            t = jnp.where(kio == k, sh_as[j - k], 0.0)
            ab = t if ab is None else ab + t
        s = ab + adst
        e = jnp.where(s > 0, s, 0.2 * s)
        bit_j = (mp >> j) & 1
        allow = jnp.logical_or(kio == j, (own_bit & bit_j) == 1)
        ems.append(jnp.where(allow, e, _NEG))

    mmax = functools.reduce(jnp.maximum, ems)
    exs = [jnp.exp(em - mmax) for em in ems]
    denom = functools.reduce(jnp.add, exs) + 1e-16
    alphas = [ex / denom for ex in exs]

    cols = [None] * heads
    for d in range(-3, 4):
        ks = [k for k in range(4) if 0 <= k + d < 4]
        coef = None
        for k in ks:
            t = jnp.where(kio == k, alphas[k + d], 0.0)
            coef = t if coef is None else coef + t
        hs_d = _shift(h, d)
        for i in range(heads):
            contrib = coef[:, i:i + 1] * hs_d[:, i * ch:(i + 1) * ch]
            cols[i] = contrib if cols[i] is None else cols[i] + contrib
    return cols[0] if heads == 1 else jnp.concatenate(cols, axis=1)


def _fused_kernel(x_ref, mp_ref, wv_ref, bv_ref, w1_ref, as1_ref, ad1_ref,
                  b1_ref, w2_ref, as2_ref, ad2_ref, b2_ref, fw_ref, fb_ref,
                  o_ref, *, heads, ch1, ch2):
    tr = x_ref.shape[1]
    x = x_ref[0]
    z = jnp.dot(x, wv_ref[0], preferred_element_type=jnp.float32) + bv_ref[0]
    h1 = jnp.dot(z, w1_ref[...], preferred_element_type=jnp.float32)

    mp = mp_ref[0]
    kio = jax.lax.broadcasted_iota(jnp.int32, (tr, 1), 0) & 3
    own_bit = None
    for k in range(4):
        t = jnp.where(kio == k, (mp >> k) & 1, 0)
        own_bit = t if own_bit is None else own_bit + t

    out1 = _gat_layer(h1, as1_ref[...], ad1_ref[...], kio, mp, own_bit,
                      heads, ch1)
    hmid = jnp.maximum(out1 + b1_ref[...], 0.0)
    h2 = jnp.dot(hmid, w2_ref[...], preferred_element_type=jnp.float32)
    out2 = _gat_layer(h2, as2_ref[...], ad2_ref[...], kio, mp, own_bit,
                      1, ch2)
    hf = jnp.maximum(out2 + b2_ref[...], 0.0)
    o_ref[...] = (jnp.dot(hf, fw_ref[...], preferred_element_type=jnp.float32)
                  + fb_ref[...])


def _pick_tile(b):
    best = 8
    for t in range(8, min(b, 1024) + 1, 8):
        if b % t == 0 and t % 4 == 0:
            best = t
    return best


def kernel(X, missing_pattern, view_W, view_b, W1, att_src1, att_dst1, b1,
           W2, att_src2, att_dst2, b2, fc_W, fc_b):
    V, B, in_dim = X.shape
    d_model = view_W.shape[2]
    heads, ch1 = att_src1.shape
    ch2 = att_src2.shape[1]
    out_dim = fc_W.shape[1]
    TR = _pick_tile(B)

    mpn = jnp.repeat(missing_pattern.astype(jnp.int32), 4).reshape(V, B, 1)
    bv = view_b.reshape(V, 1, d_model)
    as1 = att_src1.reshape(1, heads * ch1)
    ad1 = att_dst1.reshape(1, heads * ch1)
    b1r = b1.reshape(1, heads * ch1)
    as2 = att_src2.reshape(1, ch2)
    ad2 = att_dst2.reshape(1, ch2)
    b2r = b2.reshape(1, ch2)
    fbr = fc_b.reshape(1, out_dim)

    grid = (V, B // TR)
    fixed = lambda v, c: (0, 0)
    out2d = pl.pallas_call(
        functools.partial(_fused_kernel, heads=heads, ch1=ch1, ch2=ch2),
        grid=grid,
        in_specs=[
            pl.BlockSpec((1, TR, in_dim), lambda v, c: (v, c, 0)),
            pl.BlockSpec((1, TR, 1), lambda v, c: (v, c, 0)),
            pl.BlockSpec((1, in_dim, d_model), lambda v, c: (v, 0, 0)),
            pl.BlockSpec((1, 1, d_model), lambda v, c: (v, 0, 0)),
            pl.BlockSpec(W1.shape, fixed),
            pl.BlockSpec(as1.shape, fixed),
            pl.BlockSpec(ad1.shape, fixed),
            pl.BlockSpec(b1r.shape, fixed),
            pl.BlockSpec(W2.shape, fixed),
            pl.BlockSpec(as2.shape, fixed),
            pl.BlockSpec(ad2.shape, fixed),
            pl.BlockSpec(b2r.shape, fixed),
            pl.BlockSpec(fc_W.shape, fixed),
            pl.BlockSpec(fbr.shape, fixed),
        ],
        out_specs=pl.BlockSpec((TR, out_dim), lambda v, c: (c, v)),
        out_shape=jax.ShapeDtypeStruct((B, V * out_dim), jnp.float32),
    )(X, mpn, view_W, bv, W1, as1, ad1, b1r, W2, as2, ad2, b2r, fc_W, fbr)
    return out2d.reshape(B, V, out_dim)


# speedup vs baseline: 63.1947x; 1.9405x over previous
import functools

import jax
import jax.numpy as jnp
from jax.experimental import pallas as pl

_NEG = -1e30


def _shift(a, d):
    if d == 0:
        return a
    return jnp.concatenate([a[d:], a[:d]], axis=0)


def _gat_layer(h, att_mat, kio, own_bit, heads, ch):
    tr = h.shape[0]
    a = jnp.dot(h, att_mat, preferred_element_type=jnp.float32)
    asrc, adst = a[:, :heads], a[:, heads:]

    ems = {}
    for d in range(-3, 4):
        s = _shift(asrc, d) + adst
        e = jnp.where(s > 0, s, 0.2 * s)
        if d == 0:
            allow = None
        else:
            kd = kio + d
            inr = jnp.logical_and(kd >= 0, kd <= 3)
            allow = jnp.logical_and(inr, (own_bit & _shift(own_bit, d)) == 1)
        ems[d] = e if allow is None else jnp.where(allow, e, _NEG)

    mmax = functools.reduce(jnp.maximum, ems.values())
    exs = {d: jnp.exp(em - mmax) for d, em in ems.items()}
    denom = functools.reduce(jnp.add, exs.values()) + 1e-16
    inv = 1.0 / denom

    if heads > 1:
        rep = (jax.lax.broadcasted_iota(jnp.int32, (heads, heads * ch), 1)
               // ch == jax.lax.broadcasted_iota(
                   jnp.int32, (heads, heads * ch), 0)).astype(jnp.float32)
    out = None
    for d in range(-3, 4):
        coef = exs[d] * inv
        if heads > 1:
            coef = jnp.dot(coef, rep, preferred_element_type=jnp.float32)
        contrib = coef * _shift(h, d)
        out = contrib if out is None else out + contrib
    return out


def _fused_kernel(x_ref, mp_ref, wv_ref, bv_ref, w1_ref, am1_ref,
                  b1_ref, w2_ref, am2_ref, b2_ref, fw_ref, fb_ref,
                  o_ref, *, heads, ch1, ch2):
    tr = x_ref.shape[1]
    x = x_ref[0]
    z = jnp.dot(x, wv_ref[0], preferred_element_type=jnp.float32) + bv_ref[0]
    h1 = jnp.dot(z, w1_ref[...], preferred_element_type=jnp.float32)

    mp = mp_ref[0]
    kio = jax.lax.broadcasted_iota(jnp.int32, (tr, 1), 0) & 3
    own_bit = None
    for k in range(4):
        t = jnp.where(kio == k, (mp >> k) & 1, 0)
        own_bit = t if own_bit is None else own_bit + t

    out1 = _gat_layer(h1, am1_ref[...], kio, own_bit, heads, ch1)
    hmid = jnp.maximum(out1 + b1_ref[...], 0.0)
    h2 = jnp.dot(hmid, w2_ref[...], preferred_element_type=jnp.float32)
    out2 = _gat_layer(h2, am2_ref[...], kio, own_bit, 1, ch2)
    hf = jnp.maximum(out2 + b2_ref[...], 0.0)
    o_ref[...] = (jnp.dot(hf, fw_ref[...], preferred_element_type=jnp.float32)
                  + fb_ref[...])


def _pick_tile(b):
    best = 8
    for t in range(8, min(b, 1024) + 1, 8):
        if b % t == 0 and t % 4 == 0:
            best = t
    return best


def kernel(X, missing_pattern, view_W, view_b, W1, att_src1, att_dst1, b1,
           W2, att_src2, att_dst2, b2, fc_W, fc_b):
    V, B, in_dim = X.shape
    d_model = view_W.shape[2]
    heads, ch1 = att_src1.shape
    ch2 = att_src2.shape[1]
    out_dim = fc_W.shape[1]
    TR = _pick_tile(B)

    mpn = jnp.repeat(missing_pattern.astype(jnp.int32), 4).reshape(V, B, 1)
    bv = view_b.reshape(V, 1, d_model)
    b1r = b1.reshape(1, heads * ch1)
    b2r = b2.reshape(1, ch2)
    fbr = fc_b.reshape(1, out_dim)

    def att_matrix(a_s, a_d):
        nh, c = a_s.shape
        eye = jnp.eye(nh, dtype=jnp.float32)
        left = (a_s[:, :, None] * eye[:, None, :]).reshape(nh * c, nh)
        right = (a_d[:, :, None] * eye[:, None, :]).reshape(nh * c, nh)
        return jnp.concatenate([left, right], axis=1)

    am1 = att_matrix(att_src1, att_dst1)
    am2 = att_matrix(att_src2, att_dst2)

    grid = (V, B // TR)
    fixed = lambda v, c: (0, 0)
    out2d = pl.pallas_call(
        functools.partial(_fused_kernel, heads=heads, ch1=ch1, ch2=ch2),
        grid=grid,
        in_specs=[
            pl.BlockSpec((1, TR, in_dim), lambda v, c: (v, c, 0)),
            pl.BlockSpec((1, TR, 1), lambda v, c: (v, c, 0)),
            pl.BlockSpec((1, in_dim, d_model), lambda v, c: (v, 0, 0)),
            pl.BlockSpec((1, 1, d_model), lambda v, c: (v, 0, 0)),
            pl.BlockSpec(W1.shape, fixed),
            pl.BlockSpec(am1.shape, fixed),
            pl.BlockSpec(b1r.shape, fixed),
            pl.BlockSpec(W2.shape, fixed),
            pl.BlockSpec(am2.shape, fixed),
            pl.BlockSpec(b2r.shape, fixed),
            pl.BlockSpec(fc_W.shape, fixed),
            pl.BlockSpec(fbr.shape, fixed),
        ],
        out_specs=pl.BlockSpec((TR, out_dim), lambda v, c: (c, v)),
        out_shape=jax.ShapeDtypeStruct((B, V * out_dim), jnp.float32),
    )(X, mpn, view_W, bv, W1, am1, b1r, W2, am2, b2r, fc_W, fbr)
    return out2d.reshape(B, V, out_dim)
